# baseline (speedup 1.0000x reference)
import functools

import jax
import jax.numpy as jnp
from jax import lax
from jax.experimental import pallas as pl
from jax.experimental.pallas import tpu as pltpu
from jax.experimental.pallas import tpu_sc as plsc

_BATCH = 16384
_C = 32
_NW = 32
_BPW = _BATCH // _NW
_LN2 = 0.6931471805599453


def _ln(x):
    bits = lax.bitcast_convert_type(x, jnp.int32)
    e = jnp.right_shift(bits, 23) - 127
    m_bits = jnp.bitwise_or(jnp.bitwise_and(bits, 0x007FFFFF), 0x3F800000)
    m = lax.bitcast_convert_type(m_bits, jnp.float32)
    s = (m - 1.0) / (m + 1.0)
    s2 = s * s
    p = 2.0 * s * (1.0 + s2 * (1.0 / 3.0 + s2 * (0.2 + s2 * (1.0 / 7.0 + s2 * (1.0 / 9.0)))))
    return e.astype(jnp.float32) * _LN2 + p


def _sigmoid04(x):
    return 0.4 / (1.0 + jnp.exp(-x))


def _body(uid_hbm, qid_hbm, theta_hbm, slip_hbm, guess_hbm, qtab_hbm, out_hbm,
          uid_v, qid_v, theta_v, qtab_v, slip_v, guess_v, out_v,
          sem0, sem1, sem2, sem3):
    wid = lax.axis_index("s") * 2 + lax.axis_index("c")
    base = wid * _BPW

    pltpu.sync_copy(uid_hbm.at[pl.ds(base, _BPW)], uid_v)
    pltpu.sync_copy(qid_hbm.at[pl.ds(base, _BPW)], qid_v)

    cp0 = pltpu.async_copy(theta_hbm.at[uid_v], theta_v, sem0)
    cp1 = pltpu.async_copy(qtab_hbm.at[qid_v], qtab_v, sem1)
    cp2 = pltpu.async_copy(slip_hbm.at[qid_v], slip_v, sem2)
    cp3 = pltpu.async_copy(guess_hbm.at[qid_v], guess_v, sem3)
    cp0.wait()
    cp1.wait()
    cp2.wait()
    cp3.wait()

    lanes = lax.iota(jnp.int32, 16)

    def chunk(i, carry):
        rows = i * 16 + lanes
        n = jnp.full((16,), 1.0, jnp.float32)
        for c in range(_C):
            col = jnp.full((16,), c, jnp.int32)
            t = plsc.load_gather(theta_v, [rows, col])
            q = plsc.load_gather(qtab_v, [rows, col])
            bad = jnp.logical_and(q > 0.5, t <= 0.0)
            n = n * jnp.where(bad, 0.5, 1.0)
        sraw = slip_v[pl.ds(i * 16, 16)]
        graw = guess_v[pl.ds(i * 16, 16)]
        a = 1.0 - _sigmoid04(sraw)
        g = jnp.maximum(_sigmoid04(graw), 1e-30)
        out = jnp.exp(n * _ln(a) + (1.0 - n) * _ln(g))
        out_v[pl.ds(i * 16, 16)] = out
        return carry

    lax.fori_loop(0, _BPW // 16, chunk, 0)
    pltpu.sync_copy(out_v, out_hbm.at[pl.ds(base, _BPW)])


@jax.jit
def _dina_sc(uid, qid, theta_w, slip_w, guess_w, q_table):
    run = pl.kernel(
        _body,
        out_type=jax.ShapeDtypeStruct((_BATCH,), jnp.float32),
        mesh=plsc.VectorSubcoreMesh(core_axis_name="c", subcore_axis_name="s"),
        compiler_params=pltpu.CompilerParams(
            needs_layout_passes=False, use_tc_tiling_on_sc=False
        ),
        scratch_types=[
            pltpu.VMEM((_BPW,), jnp.int32),
            pltpu.VMEM((_BPW,), jnp.int32),
            pltpu.VMEM((_BPW, _C), jnp.float32),
            pltpu.VMEM((_BPW, _C), jnp.float32),
            pltpu.VMEM((_BPW,), jnp.float32),
            pltpu.VMEM((_BPW,), jnp.float32),
            pltpu.VMEM((_BPW,), jnp.float32),
            pltpu.SemaphoreType.DMA,
            pltpu.SemaphoreType.DMA,
            pltpu.SemaphoreType.DMA,
            pltpu.SemaphoreType.DMA,
        ],
    )
    return run(uid, qid, theta_w, slip_w, guess_w, q_table)


def kernel(user_id, question_id, theta_w, slip_w, guess_w, q_table):
    return _dina_sc(
        user_id.astype(jnp.int32),
        question_id.astype(jnp.int32),
        theta_w,
        slip_w.reshape(-1),
        guess_w.reshape(-1),
        q_table,
    )
